# speedup vs baseline: 1.0003x; 1.0003x over previous
import functools

import jax
import jax.numpy as jnp
from jax.experimental import pallas as pl
from jax.experimental.pallas import tpu as pltpu


def _round_up(x, m):
    return (x + m - 1) // m * m


def _fwd_kernel(idx_ref, tgt_ref, table_ref, logp_ref, logits_ref, loss_ref,
                *, n_actual):
    tn, vp = logits_ref.shape
    idx = idx_ref[...]
    tgt = tgt_ref[...]
    cols = jax.lax.broadcasted_iota(jnp.int32, (tn, vp), 1)

    onehot = (cols == idx).astype(jnp.bfloat16)
    logits_ref[...] = jnp.dot(onehot, table_ref[...],
                              preferred_element_type=jnp.float32)

    logp = jnp.dot(onehot, logp_ref[...],
                   preferred_element_type=jnp.float32)
    picked = jnp.sum(jnp.where(cols == tgt, logp, 0.0),
                     axis=-1, keepdims=True)

    if n_actual is None:
        loss_ref[...] = -picked
    else:
        pid = pl.program_id(0)
        rows = pid * tn + jax.lax.broadcasted_iota(jnp.int32, (tn, 1), 0)
        loss_ref[...] = jnp.where(rows < n_actual, -picked, 0.0)


def kernel(idx, targets, table_padded, logp_padded):
    B, T = idx.shape
    v = 256
    vp = table_padded.shape[1]
    n = B * T
    tn = 2048

    idx_flat = idx.reshape(n, 1).astype(jnp.int32)
    tgt_flat = targets.reshape(n, 1).astype(jnp.int32)
    table_bf = table_padded.astype(jnp.bfloat16)
    logp_bf = logp_padded.astype(jnp.bfloat16)

    n_pad = _round_up(n, tn)
    pad = n_pad - n
    if pad:
        idx_flat = jnp.pad(idx_flat, ((0, pad), (0, 0)))
        tgt_flat = jnp.pad(tgt_flat, ((0, pad), (0, 0)))

    logits_p, loss_rows = pl.pallas_call(
        functools.partial(_fwd_kernel, n_actual=(None if pad == 0 else n)),
        out_shape=(
            jax.ShapeDtypeStruct((n_pad, vp), jnp.float32),
            jax.ShapeDtypeStruct((n_pad, 1), jnp.float32),
        ),
        grid_spec=pltpu.PrefetchScalarGridSpec(
            num_scalar_prefetch=0,
            grid=(n_pad // tn,),
            in_specs=[
                pl.BlockSpec((tn, 1), lambda i: (i, 0)),
                pl.BlockSpec((tn, 1), lambda i: (i, 0)),
                pl.BlockSpec((vp, vp), lambda i: (0, 0)),
                pl.BlockSpec((vp, vp), lambda i: (0, 0)),
            ],
            out_specs=(
                pl.BlockSpec((tn, vp), lambda i: (i, 0)),
                pl.BlockSpec((tn, 1), lambda i: (i, 0)),
            ),
        ),
        compiler_params=pltpu.CompilerParams(
            dimension_semantics=("parallel",)),
    )(idx_flat, tgt_flat, table_bf, logp_bf)

    logits = jax.lax.slice(logits_p, (0, 0), (n, v))
    loss = jnp.sum(loss_rows) * (1.0 / n)
    return logits, loss


# speedup vs baseline: 8.4925x; 8.4900x over previous
import jax
import jax.numpy as jnp
from jax.experimental import pallas as pl
from jax.experimental.pallas import tpu as pltpu


def _fwd_kernel(idx_ref, tgt_ref, table_ref, logp_ref, out_ref, loss_ref):
    bb, tc = idx_ref.shape
    vp = table_ref.shape[0]
    idx = idx_ref[...]
    tgt = tgt_ref[...]
    table = table_ref[...]
    logp = logp_ref[...]
    viota = jax.lax.broadcasted_iota(jnp.int32, (vp, tc), 0)

    acc = jnp.zeros((1, vp), jnp.float32)
    for r in range(bb):
        idx_r = jax.lax.slice(idx, (r, 0), (r + 1, tc))
        tgt_r = jax.lax.slice(tgt, (r, 0), (r + 1, tc))
        a = (viota == idx_r).astype(jnp.bfloat16)
        b = (viota == tgt_r).astype(jnp.bfloat16)
        out_ref[r] = jax.lax.dot_general(
            a, table, (((0,), (0,)), ((), ())),
            preferred_element_type=jnp.float32)
        counts = jax.lax.dot_general(
            a, b, (((1,), (1,)), ((), ())),
            preferred_element_type=jnp.float32)
        acc = acc + jnp.sum(counts * logp, axis=0, keepdims=True)
    loss_ref[...] = acc.reshape(1, 1, 1, vp)


def kernel(idx, targets, table_padded, logp_padded):
    B, T = idx.shape
    v = 256
    vp = table_padded.shape[1]
    n = B * T
    bb = 8
    tc = 1024
    assert B % bb == 0 and T % tc == 0

    idx = idx.astype(jnp.int32)
    tgt = targets.astype(jnp.int32)
    table_bf = table_padded.astype(jnp.bfloat16)

    gb, gt = B // bb, T // tc
    logits3, loss_parts = pl.pallas_call(
        _fwd_kernel,
        out_shape=(
            jax.ShapeDtypeStruct((B, T, vp), jnp.float32),
            jax.ShapeDtypeStruct((gb, gt, 1, vp), jnp.float32),
        ),
        grid_spec=pltpu.PrefetchScalarGridSpec(
            num_scalar_prefetch=0,
            grid=(gb, gt),
            in_specs=[
                pl.BlockSpec((bb, tc), lambda i, j: (i, j)),
                pl.BlockSpec((bb, tc), lambda i, j: (i, j)),
                pl.BlockSpec((vp, vp), lambda i, j: (0, 0)),
                pl.BlockSpec((vp, vp), lambda i, j: (0, 0)),
            ],
            out_specs=(
                pl.BlockSpec((bb, tc, vp), lambda i, j: (i, j, 0)),
                pl.BlockSpec((1, 1, 1, vp), lambda i, j: (i, j, 0, 0)),
            ),
        ),
        compiler_params=pltpu.CompilerParams(
            dimension_semantics=("parallel", "parallel")),
    )(idx, tgt, table_bf, logp_padded)

    logits = logits3.reshape(n, vp)
    if vp != v:
        logits = jax.lax.slice(logits, (0, 0), (n, v))
    loss = -jnp.sum(loss_parts) * (1.0 / n)
    return logits, loss


# speedup vs baseline: 9.2027x; 1.0836x over previous
import jax
import jax.numpy as jnp
from jax.experimental import pallas as pl
from jax.experimental.pallas import tpu as pltpu


def _fwd_kernel(idx_ref, tgt_ref, table_ref, logp_ref, out_ref, loss_ref):
    bb, tc = idx_ref.shape
    vp = table_ref.shape[0]
    idx = idx_ref[...]
    tgt = tgt_ref[...]
    table = table_ref[...]
    logp = logp_ref[...]
    viota = jax.lax.broadcasted_iota(jnp.int32, (vp, tc), 0)

    acc = jnp.zeros((1, vp), jnp.float32)
    for r in range(bb):
        idx_r = jax.lax.slice(idx, (r, 0), (r + 1, tc))
        tgt_r = jax.lax.slice(tgt, (r, 0), (r + 1, tc))
        a = (viota == idx_r).astype(jnp.bfloat16)
        b = (viota == tgt_r).astype(jnp.bfloat16)
        out_ref[r] = jax.lax.dot_general(
            a, table, (((0,), (0,)), ((), ())),
            preferred_element_type=jnp.float32)
        counts = jax.lax.dot_general(
            a, b, (((1,), (1,)), ((), ())),
            preferred_element_type=jnp.float32)
        acc = acc + jnp.sum(counts * logp, axis=0, keepdims=True)
    loss_ref[...] = acc.reshape(1, 1, 1, vp)


def kernel(idx, targets, table_padded, logp_padded):
    B, T = idx.shape
    v = 256
    vp = table_padded.shape[1]
    n = B * T
    bb = 8
    tc = 2048
    assert B % bb == 0 and T % tc == 0

    idx = idx.astype(jnp.int32)
    tgt = targets.astype(jnp.int32)
    table_bf = table_padded.astype(jnp.bfloat16)

    gb, gt = B // bb, T // tc
    logits3, loss_parts = pl.pallas_call(
        _fwd_kernel,
        out_shape=(
            jax.ShapeDtypeStruct((B, T, vp), jnp.float32),
            jax.ShapeDtypeStruct((gb, gt, 1, vp), jnp.float32),
        ),
        grid_spec=pltpu.PrefetchScalarGridSpec(
            num_scalar_prefetch=0,
            grid=(gb, gt),
            in_specs=[
                pl.BlockSpec((bb, tc), lambda i, j: (i, j)),
                pl.BlockSpec((bb, tc), lambda i, j: (i, j)),
                pl.BlockSpec((vp, vp), lambda i, j: (0, 0)),
                pl.BlockSpec((vp, vp), lambda i, j: (0, 0)),
            ],
            out_specs=(
                pl.BlockSpec((bb, tc, vp), lambda i, j: (i, j, 0)),
                pl.BlockSpec((1, 1, 1, vp), lambda i, j: (i, j, 0, 0)),
            ),
        ),
        compiler_params=pltpu.CompilerParams(
            dimension_semantics=("parallel", "parallel")),
    )(idx, tgt, table_bf, logp_padded)

    logits = logits3.reshape(n, vp)
    if vp != v:
        logits = jax.lax.slice(logits, (0, 0), (n, v))
    loss = -jnp.sum(loss_parts) * (1.0 / n)
    return logits, loss
